# baseline (speedup 1.0000x reference)
import functools
import jax
import jax.numpy as jnp
from jax import lax
from jax.experimental import pallas as pl
from jax.experimental.pallas import tpu as pltpu, tpu_sc as plsc

NC, NS, L = 2, 16, 16
NW = NC * NS
NB = 4096
NP = 12
BPW = NB // NW
NBUF = 5
LOOK = 2

_mesh = plsc.VectorSubcoreMesh(core_axis_name="c", subcore_axis_name="s")


@functools.partial(
    pl.kernel,
    out_type=jax.ShapeDtypeStruct((NP, NB, 768), jnp.float32),
    mesh=_mesh,
    scratch_types=[
        pltpu.VMEM((BPW * NP * 7 // 2,), jnp.int32),
        pltpu.VMEM((NP * 7, BPW), jnp.int32),
        pltpu.VMEM((NBUF, BPW, 128), jnp.float32),
        pltpu.VMEM((1, BPW, 128), jnp.float32),
        pltpu.SemaphoreType.DMA((NBUF,)),
        pltpu.SemaphoreType.DMA((NBUF,)),
        pltpu.SemaphoreType.DMA,
        pltpu.SemaphoreType.DMA,
        pltpu.VMEM_SHARED((1024, 128), jnp.float32),
        pltpu.VMEM_SHARED((352, 128), jnp.float32),
    ],
    compiler_params=pltpu.CompilerParams(
        use_tc_tiling_on_sc=True, needs_layout_passes=False),
)
def _embed(sp_hbm, mv_hbm, ab_hbm, it_hbm, ids_hbm, out_hbm,
           ids_v, idx_v, ring, ibuf, gsems, wsems, isem, ssem,
           mv_s, ab_s):
    wid = lax.axis_index("s") * NC + lax.axis_index("c")
    sid = lax.axis_index("s")
    bat0 = wid * BPW
    pltpu.async_copy(ids_hbm.at[wid], ids_v, ssem).wait()
    lane = lax.iota(jnp.int32, L)

    pltpu.sync_copy(mv_hbm.at[pl.ds(sid * 64, 64)], ring.at[0, pl.ds(0, 64)])
    pltpu.sync_copy(ring.at[0, pl.ds(0, 64)], mv_s.at[pl.ds(sid * 64, 64)])

    @pl.when(sid < 11)
    def _stage_ab():
        pltpu.sync_copy(ab_hbm.at[pl.ds(sid * 32, 32)], ring.at[1, pl.ds(0, 32)])
        pltpu.sync_copy(ring.at[1, pl.ds(0, 32)], ab_s.at[pl.ds(sid * 32, 32)])
    plsc.subcore_barrier()

    @pl.loop(0, BPW // L)
    def _build(g):
        base = g * L
        pos = (base + lane) * (NP * 7 // 2)
        for r in range(NP * 7):
            w = plsc.load_gather(ids_v, [pos + r // 2])
            if r % 2:
                w = lax.shift_right_logical(w, 16)
            idx_v[r, pl.ds(base, L)] = lax.bitwise_and(w, 0xFFFF)

    def _gdesc(tbl, r, b):
        return pltpu.make_async_copy(
            tbl.at[idx_v.at[r]], ring.at[b], gsems.at[b])

    def _idesc(k):
        return pltpu.make_async_copy(
            it_hbm.at[idx_v.at[k * 7 + 6]], ibuf.at[0], isem)

    def _wdesc(p, col, b):
        return pltpu.make_async_copy(
            ring.at[b],
            out_hbm.at[p, pl.ds(bat0, BPW), pl.ds(col * 128, 128)],
            wsems.at[b])

    _idesc(0).start()

    def _run_phase(steps, gfn, wfn, prev, body=None):
        def _prev_wait(j):
            p, col = prev[0](j)
            _wdesc(p, col, j % NBUF if isinstance(j, int) else lax.rem(j, NBUF)).wait()

        for k in range(LOOK):
            if prev is not None:
                j0 = prev[1] - NBUF
                _prev_wait(j0 + ((k - j0 % NBUF) % NBUF))
            gfn(k, k).start()

        @pl.loop(0, steps)
        def _step(k):
            b = lax.rem(k, NBUF)
            gfn(k, b).wait()
            if body is not None:
                body(k, b)
            p, col = wfn(k)
            _wdesc(p, col, b).start()
            g = k + LOOK

            @pl.when(g < steps)
            def _():
                bg = lax.rem(g, NBUF)

                @pl.when(g >= NBUF)
                def _():
                    pg, colg = wfn(g - NBUF)
                    _wdesc(pg, colg, bg).wait()

                if prev is not None:
                    @pl.when(g < NBUF)
                    def _():
                        j0 = prev[1] - NBUF
                        off = lax.rem(g - (j0 % NBUF) + 2 * NBUF, NBUF)
                        _prev_wait(j0 + off)
                gfn(g, bg).start()

    sp_wfn = lambda k: (k, 0)
    _run_phase(NP,
               lambda k, b: _gdesc(sp_hbm, k * 7, b),
               sp_wfn, None)

    mv_wfn = lambda k: (k // 4, 1 + lax.rem(k, 4))
    _run_phase(4 * NP,
               lambda k, b: _gdesc(mv_s, (k // 4) * 7 + 1 + lax.rem(k, 4), b),
               mv_wfn, (sp_wfn, NP))

    def _abit_body(k, b):
        _idesc(k).wait()

        @pl.loop(0, BPW)
        def _merge(r):
            for q in range(4):
                ring[b, r, pl.ds(64 + q * L, L)] = ibuf[0, r, pl.ds(64 + q * L, L)]

        @pl.when(k + 1 < NP)
        def _():
            _idesc(k + 1).start()

    ab_wfn = lambda k: (k, 5)
    _run_phase(NP,
               lambda k, b: _gdesc(ab_s, k * 7 + 5, b),
               ab_wfn, (mv_wfn, 4 * NP), body=_abit_body)

    for d in range(NBUF):
        k = NP - NBUF + d
        _wdesc(k, 5, k % NBUF).wait()


def kernel(int_ids, species_table, move_table, ability_table, item_table):
    ids16 = int_ids.astype(jnp.int16).reshape(NW, BPW * NP * 7 // 2, 2)
    ids = jax.lax.bitcast_convert_type(ids16, jnp.int32)
    ab_p = jnp.pad(ability_table, ((0, 2), (0, 64)))
    it_p = jnp.pad(item_table, ((0, 24), (64, 0)))
    mv_p = jnp.pad(move_table, ((0, 24), (0, 0)))
    out = _embed(species_table, mv_p, ab_p, it_p, ids)
    return jnp.transpose(out, (1, 0, 2))


# speedup vs baseline: 1.5176x; 1.5176x over previous
import functools
import jax
import jax.numpy as jnp
from jax import lax
from jax.experimental import pallas as pl
from jax.experimental.pallas import tpu as pltpu, tpu_sc as plsc

NC, NS, L = 2, 16, 16
NW = NC * NS
NB = 4096
NP = 12
BPW = NB // NW
NBUF = 5
LOOK = 2
SPP = 6
NSTEP = NP * SPP

_mesh = plsc.VectorSubcoreMesh(core_axis_name="c", subcore_axis_name="s")


@functools.partial(
    pl.kernel,
    out_type=jax.ShapeDtypeStruct((NP, NB, 768), jnp.float32),
    mesh=_mesh,
    scratch_types=[
        pltpu.VMEM((BPW * NP * 7,), jnp.int32),
        pltpu.VMEM((NP * 7, BPW), jnp.int32),
        pltpu.VMEM((NBUF, BPW, 128), jnp.float32),
        pltpu.VMEM((1, BPW, 128), jnp.float32),
        pltpu.SemaphoreType.DMA((NBUF,)),
        pltpu.SemaphoreType.DMA((NBUF,)),
        pltpu.SemaphoreType.DMA,
        pltpu.SemaphoreType.DMA,
        pltpu.VMEM_SHARED((1024, 128), jnp.float32),
    ],
    compiler_params=pltpu.CompilerParams(
        use_tc_tiling_on_sc=True, needs_layout_passes=False),
)
def _embed(sp_hbm, mv_hbm, ab_hbm, it_hbm, ids_hbm, out_hbm,
           ids_v, idx_v, ring, ibuf, gsems, wsems, isem, ssem, mv_s):
    wid = lax.axis_index("s") * NC + lax.axis_index("c")
    sid = lax.axis_index("s")
    bat0 = wid * BPW
    lane = lax.iota(jnp.int32, L)

    pltpu.sync_copy(mv_hbm.at[pl.ds(sid * 64, 64)], ring.at[0, pl.ds(0, 64)])
    pltpu.sync_copy(ring.at[0, pl.ds(0, 64)], mv_s.at[pl.ds(sid * 64, 64)])

    pltpu.async_copy(ids_hbm.at[wid], ids_v, ssem).wait()

    @pl.loop(0, BPW // L)
    def _build(g):
        base = g * L
        pos = (base + lane) * (NP * 7)
        for r in range(NP * 7):
            idx_v[r, pl.ds(base, L)] = plsc.load_gather(ids_v, [pos + r])

    plsc.subcore_barrier()

    def _gdesc(j, p, b):
        if j == 0:
            tbl, r = sp_hbm, p * 7
        elif j <= 4:
            tbl, r = mv_s, p * 7 + j
        else:
            tbl, r = ab_hbm, p * 7 + 5
        return pltpu.make_async_copy(
            tbl.at[idx_v.at[r]], ring.at[b], gsems.at[b])

    def _idesc(p):
        return pltpu.make_async_copy(
            it_hbm.at[idx_v.at[p * 7 + 6]], ibuf.at[0], isem)

    def _wdesc(p, col, b):
        return pltpu.make_async_copy(
            ring.at[b],
            out_hbm.at[p, pl.ds(bat0, BPW), pl.ds(col * 128, 128)],
            wsems.at[b])

    _idesc(0).start()
    for k in range(LOOK):
        _gdesc(k, 0, k).start()

    @pl.loop(0, NP)
    def _party(p):
        k0 = p * SPP
        for j in range(SPP):
            k = k0 + j
            b = lax.rem(k, NBUF)
            _gdesc(j, p, b).wait()
            if j == 5:
                _idesc(p).wait()

                @pl.loop(0, BPW)
                def _merge(r):
                    for q in range(4):
                        ring[b, r, pl.ds(64 + q * L, L)] = (
                            ibuf[0, r, pl.ds(64 + q * L, L)])

                @pl.when(p + 1 < NP)
                def _():
                    _idesc(p + 1).start()
            _wdesc(p, j, b).start()

            g = k + LOOK
            jg = (j + LOOK) % SPP
            pg = p + (j + LOOK) // SPP

            @pl.when(g < NSTEP)
            def _():
                bg = lax.rem(g, NBUF)

                @pl.when(g >= NBUF)
                def _():
                    jw = (j + LOOK - NBUF) % SPP
                    pw = p - 1 if j + LOOK - NBUF < 0 else p
                    _wdesc(pw, jw, bg).wait()
                _gdesc(jg, pg, bg).start()

    for d in range(NBUF):
        k = NSTEP - NBUF + d
        _wdesc(NP - 1, k % SPP, k % NBUF).wait()


def kernel(int_ids, species_table, move_table, ability_table, item_table):
    ids = int_ids.astype(jnp.int32).reshape(NW, BPW * NP * 7)
    ab_p = jnp.pad(ability_table, ((0, 2), (0, 64)))
    it_p = jnp.pad(item_table, ((0, 24), (64, 0)))
    mv_p = jnp.pad(move_table, ((0, 24), (0, 0)))
    out = _embed(species_table, mv_p, ab_p, it_p, ids)
    return jnp.transpose(out, (1, 0, 2))


# speedup vs baseline: 1.5415x; 1.0158x over previous
import functools
import jax
import jax.numpy as jnp
from jax import lax
from jax.experimental import pallas as pl
from jax.experimental.pallas import tpu as pltpu, tpu_sc as plsc

NC, NS, L = 2, 16, 16
NW = NC * NS
NB = 4096
NP = 12
BPW = NB // NW
NBUF = 5
LOOK = 3
SPP = 6
NSTEP = NP * SPP

_mesh = plsc.VectorSubcoreMesh(core_axis_name="c", subcore_axis_name="s")


@functools.partial(
    pl.kernel,
    out_type=jax.ShapeDtypeStruct((NP, NB, 768), jnp.float32),
    mesh=_mesh,
    scratch_types=[
        pltpu.VMEM((BPW * NP * 7,), jnp.int32),
        pltpu.VMEM((NP * 7, BPW), jnp.int32),
        pltpu.VMEM((NBUF, BPW, 128), jnp.float32),
        pltpu.VMEM((1, BPW, 128), jnp.float32),
        pltpu.SemaphoreType.DMA((NBUF,)),
        pltpu.SemaphoreType.DMA((NBUF,)),
        pltpu.SemaphoreType.DMA,
        pltpu.SemaphoreType.DMA,
        pltpu.VMEM_SHARED((1024, 128), jnp.float32),
    ],
    compiler_params=pltpu.CompilerParams(
        use_tc_tiling_on_sc=True, needs_layout_passes=False),
)
def _embed(sp_hbm, mv_hbm, ab_hbm, it_hbm, ids_hbm, out_hbm,
           ids_v, idx_v, ring, ibuf, gsems, wsems, isem, ssem, mv_s):
    wid = lax.axis_index("s") * NC + lax.axis_index("c")
    sid = lax.axis_index("s")
    bat0 = wid * BPW
    lane = lax.iota(jnp.int32, L)

    pltpu.sync_copy(mv_hbm.at[pl.ds(sid * 64, 64)], ring.at[0, pl.ds(0, 64)])
    pltpu.sync_copy(ring.at[0, pl.ds(0, 64)], mv_s.at[pl.ds(sid * 64, 64)])

    pltpu.async_copy(ids_hbm.at[wid], ids_v, ssem).wait()

    @pl.loop(0, BPW // L)
    def _build(g):
        base = g * L
        pos = (base + lane) * (NP * 7)
        for r in range(NP * 7):
            idx_v[r, pl.ds(base, L)] = plsc.load_gather(ids_v, [pos + r])

    plsc.subcore_barrier()

    def _gdesc(j, p, b):
        if j == 0:
            tbl, r = sp_hbm, p * 7
        elif j <= 4:
            tbl, r = mv_s, p * 7 + j
        else:
            tbl, r = ab_hbm, p * 7 + 5
        return pltpu.make_async_copy(
            tbl.at[idx_v.at[r]], ring.at[b], gsems.at[b])

    def _idesc(p):
        return pltpu.make_async_copy(
            it_hbm.at[idx_v.at[p * 7 + 6]], ibuf.at[0], isem)

    def _wdesc(p, col, b):
        return pltpu.make_async_copy(
            ring.at[b],
            out_hbm.at[p, pl.ds(bat0, BPW), pl.ds(col * 128, 128)],
            wsems.at[b])

    _idesc(0).start()
    for k in range(LOOK):
        _gdesc(k, 0, k).start()

    @pl.loop(0, NP)
    def _party(p):
        k0 = p * SPP
        for j in range(SPP):
            k = k0 + j
            b = lax.rem(k, NBUF)
            _gdesc(j, p, b).wait()
            if j == 5:
                _idesc(p).wait()

                @pl.loop(0, BPW)
                def _merge(r):
                    for q in range(4):
                        ring[b, r, pl.ds(64 + q * L, L)] = (
                            ibuf[0, r, pl.ds(64 + q * L, L)])

                @pl.when(p + 1 < NP)
                def _():
                    _idesc(p + 1).start()
            _wdesc(p, j, b).start()

            g = k + LOOK
            jg = (j + LOOK) % SPP
            pg = p + (j + LOOK) // SPP

            @pl.when(g < NSTEP)
            def _():
                bg = lax.rem(g, NBUF)

                @pl.when(g >= NBUF)
                def _():
                    jw = (j + LOOK - NBUF) % SPP
                    pw = p - 1 if j + LOOK - NBUF < 0 else p
                    _wdesc(pw, jw, bg).wait()
                _gdesc(jg, pg, bg).start()

    for d in range(NBUF):
        k = NSTEP - NBUF + d
        _wdesc(NP - 1, k % SPP, k % NBUF).wait()


def kernel(int_ids, species_table, move_table, ability_table, item_table):
    ids = int_ids.astype(jnp.int32).reshape(NW, BPW * NP * 7)
    ab_p = jnp.pad(ability_table, ((0, 2), (0, 64)))
    it_p = jnp.pad(item_table, ((0, 24), (64, 0)))
    mv_p = jnp.pad(move_table, ((0, 24), (0, 0)))
    out = _embed(species_table, mv_p, ab_p, it_p, ids)
    return jnp.transpose(out, (1, 0, 2))


# speedup vs baseline: 1.5489x; 1.0048x over previous
import functools
import jax
import jax.numpy as jnp
from jax import lax
from jax.experimental import pallas as pl
from jax.experimental.pallas import tpu as pltpu, tpu_sc as plsc

NC, NS, L = 2, 16, 16
NW = NC * NS
NB = 4096
NP = 12
BPW = NB // NW
NBUF = 4
LOOK = 2
SPP = 6
NSTEP = NP * SPP

_mesh = plsc.VectorSubcoreMesh(core_axis_name="c", subcore_axis_name="s")


@functools.partial(
    pl.kernel,
    out_type=jax.ShapeDtypeStruct((NP, NB, 768), jnp.float32),
    mesh=_mesh,
    scratch_types=[
        pltpu.VMEM((BPW * NP * 7 // 4,), jnp.int32),
        pltpu.VMEM((NP * 7, BPW), jnp.int32),
        pltpu.VMEM((NBUF, BPW, 128), jnp.float32),
        pltpu.VMEM((1, BPW, 128), jnp.float32),
        pltpu.VMEM((176, 128), jnp.float32),
        pltpu.SemaphoreType.DMA((NBUF,)),
        pltpu.SemaphoreType.DMA((NBUF,)),
        pltpu.SemaphoreType.DMA,
        pltpu.SemaphoreType.DMA,
        pltpu.VMEM_SHARED((1024, 128), jnp.float32),
    ],
    compiler_params=pltpu.CompilerParams(
        use_tc_tiling_on_sc=True, needs_layout_passes=False),
)
def _embed(sp_hbm, mv_hbm, abf_hbm, it_hbm, ids_hbm, out_hbm,
           ids_v, idx_v, ring, ibuf, ab_v, gsems, wsems, isem, ssem, mv_s):
    wid = lax.axis_index("s") * NC + lax.axis_index("c")
    sid = lax.axis_index("s")
    bat0 = wid * BPW
    lane = lax.iota(jnp.int32, L)

    pltpu.sync_copy(mv_hbm.at[pl.ds(sid * 64, 64)], ring.at[0, pl.ds(0, 64)])
    pltpu.sync_copy(ring.at[0, pl.ds(0, 64)], mv_s.at[pl.ds(sid * 64, 64)])
    pltpu.sync_copy(abf_hbm, ab_v)

    Q = BPW * NP * 7 // 4

    @pl.loop(0, 4)
    def _build(h):
        pltpu.async_copy(ids_hbm.at[wid, pl.ds(h * Q, Q)], ids_v, ssem).wait()
        for g2 in range(2):
            base = h * 2 * L + g2 * L
            pos = (g2 * L + lane) * (NP * 7)
            for r in range(NP * 7):
                idx_v[r, pl.ds(base, L)] = plsc.load_gather(ids_v, [pos + r])

    plsc.subcore_barrier()

    def _gdesc(j, p, b):
        if j == 0:
            tbl, r = sp_hbm, p * 7
        else:
            tbl, r = mv_s, p * 7 + j
        return pltpu.make_async_copy(
            tbl.at[idx_v.at[r]], ring.at[b], gsems.at[b])

    def _idesc(p):
        return pltpu.make_async_copy(
            it_hbm.at[idx_v.at[p * 7 + 6]], ibuf.at[0], isem)

    def _wdesc(p, col, b):
        return pltpu.make_async_copy(
            ring.at[b],
            out_hbm.at[p, pl.ds(bat0, BPW), pl.ds(col * 128, 128)],
            wsems.at[b])

    _idesc(0).start()
    for k in range(LOOK):
        _gdesc(k, 0, k).start()

    @pl.loop(0, NP)
    def _party(p):
        k0 = p * SPP
        for j in range(SPP):
            k = k0 + j
            b = lax.rem(k, NBUF)
            if j < 5:
                _gdesc(j, p, b).wait()
            else:
                _idesc(p).wait()
                arow = p * 7 + 5

                @pl.loop(0, BPW // L)
                def _asm(gg):
                    rbase = gg * L
                    av = idx_v[arow, pl.ds(rbase, L)]
                    for jj in range(L):
                        a = av[jj]
                        arw = lax.shift_right_logical(a, 1)
                        acl = lax.bitwise_and(a, 1) * 64
                        r = rbase + jj
                        for q in range(4):
                            ring[b, r, pl.ds(q * L, L)] = (
                                ab_v[arw, pl.ds(acl + q * L, L)])
                            ring[b, r, pl.ds(64 + q * L, L)] = (
                                ibuf[0, r, pl.ds(64 + q * L, L)])

                @pl.when(p + 1 < NP)
                def _():
                    _idesc(p + 1).start()
            _wdesc(p, j, b).start()

            g = k + LOOK
            jg = (j + LOOK) % SPP
            pg = p + (j + LOOK) // SPP

            @pl.when(g < NSTEP)
            def _():
                bg = lax.rem(g, NBUF)

                @pl.when(g >= NBUF)
                def _():
                    jw = (j + LOOK - NBUF) % SPP
                    pw = p - 1 if j + LOOK - NBUF < 0 else p
                    _wdesc(pw, jw, bg).wait()
                if jg != 5:
                    _gdesc(jg, pg, bg).start()

    for d in range(NBUF):
        k = NSTEP - NBUF + d
        _wdesc(NP - 1, k % SPP, k % NBUF).wait()


def kernel(int_ids, species_table, move_table, ability_table, item_table):
    ids = int_ids.astype(jnp.int32).reshape(NW, BPW * NP * 7)
    ab_f = jnp.pad(ability_table.reshape(175, 128), ((0, 1), (0, 0)))
    it_p = jnp.pad(item_table, ((0, 24), (64, 0)))
    mv_p = jnp.pad(move_table, ((0, 24), (0, 0)))
    out = _embed(species_table, mv_p, ab_f, it_p, ids)
    return jnp.transpose(out, (1, 0, 2))
